# Initial kernel scaffold; baseline (speedup 1.0000x reference)
import jax
import jax.numpy as jnp
from jax.experimental import pallas as pl


def kernel(h, edge_index):
    raise NotImplementedError("write your pallas kernel here")


# speedup vs baseline: 1.1989x; 1.1989x over previous
import functools

import jax
import jax.numpy as jnp
from jax import lax
from jax.experimental import pallas as pl
from jax.experimental.pallas import tpu as pltpu
from jax.experimental.pallas import tpu_sc as plsc

N_NODES = 10000
N_EDGES = 320000
D_FEAT = 128

NC = 2
NS = 16
L = 16
NW = NC * NS
EPW = N_EDGES // NW
CHUNK = 400
NCHUNK = EPW // CHUNK
NG = CHUNK // L

_mesh = plsc.VectorSubcoreMesh(core_axis_name="c", subcore_axis_name="s")


@functools.partial(
    pl.kernel,
    out_type=jax.ShapeDtypeStruct((N_EDGES,), jnp.float32),
    mesh=_mesh,
    scratch_types=[
        pltpu.VMEM((CHUNK,), jnp.int32),
        pltpu.VMEM((CHUNK,), jnp.int32),
        pltpu.VMEM((CHUNK, D_FEAT), jnp.float32),
        pltpu.VMEM((CHUNK, D_FEAT), jnp.float32),
        pltpu.VMEM((CHUNK,), jnp.float32),
        pltpu.SemaphoreType.DMA,
        pltpu.SemaphoreType.DMA,
    ],
    compiler_params=pltpu.CompilerParams(needs_layout_passes=False),
)
def _edge_dot(h_hbm, src_hbm, dst_hbm, out_hbm,
              sidx, didx, srows, drows, outv, sem_s, sem_d):
    wid = lax.axis_index("s") * NC + lax.axis_index("c")
    base = wid * EPW

    @pl.loop(0, NCHUNK)
    def _chunk(ci):
        off = base + ci * CHUNK
        pltpu.sync_copy(src_hbm.at[pl.ds(off, CHUNK)], sidx)
        pltpu.sync_copy(dst_hbm.at[pl.ds(off, CHUNK)], didx)
        cp_s = pltpu.async_copy(h_hbm.at[sidx], srows, sem_s)
        cp_d = pltpu.async_copy(h_hbm.at[didx], drows, sem_d)
        cp_s.wait()
        cp_d.wait()

        @pl.loop(0, NG)
        def _group(g):
            rows = lax.iota(jnp.int32, L) + g * L
            acc = jnp.zeros((L,), jnp.float32)
            for f in range(D_FEAT):
                fv = jnp.full((L,), f, jnp.int32)
                sv = plsc.load_gather(srows, [rows, fv])
                dv = plsc.load_gather(drows, [rows, fv])
                acc = acc + sv * dv
            outv[pl.ds(g * L, L)] = acc

        pltpu.sync_copy(outv, out_hbm.at[pl.ds(off, CHUNK)])


def kernel(h, edge_index):
    src = edge_index[0]
    dst = edge_index[1]
    return _edge_dot(h, src, dst)
